# baseline (device time: 149612 ns/iter reference)
import jax
import jax.numpy as jnp
from jax import lax
from jax.experimental import pallas as pl
from jax.experimental.pallas import tpu as pltpu

N_DEV = 4
M_BLK = 1024
K_BLK = 1024
N_TOT = 8192
N_CHUNK = 512
N_CHUNKS = N_TOT // N_CHUNK
N_TILES = N_DEV * N_CHUNKS
N_SLOTS = 4


def _body(x_ref, w_ref, out_ref, gath_ref, w_buf, send_sems, recv_sems, w_sems):
    my = lax.axis_index("i")

    d_for_phase = [None, 1, 3, 2]
    k_order = [my] + [lax.rem(my + (N_DEV - d), N_DEV) for d in d_for_phase[1:]]

    def w_dma(t):
        p, c = divmod(t, N_CHUNKS)
        return pltpu.make_async_copy(
            w_ref.at[pl.ds(k_order[p] * K_BLK, K_BLK),
                     pl.ds(c * N_CHUNK, N_CHUNK)],
            w_buf.at[t % N_SLOTS],
            w_sems.at[t % N_SLOTS],
        )

    for t in range(N_SLOTS - 1):
        w_dma(t).start()

    barrier = pltpu.get_barrier_semaphore()
    for d in range(1, N_DEV):
        peer = lax.rem(my + d, N_DEV)
        pl.semaphore_signal(
            barrier, inc=1,
            device_id=(peer,), device_id_type=pl.DeviceIdType.MESH,
        )
    pl.semaphore_wait(barrier, N_DEV - 1)

    sends = []
    for d in range(1, N_DEV):
        t = lax.rem(my + d, N_DEV)
        rdma = pltpu.make_async_remote_copy(
            src_ref=x_ref.at[t],
            dst_ref=gath_ref.at[d - 1],
            send_sem=send_sems.at[d - 1],
            recv_sem=recv_sems.at[d - 1],
            device_id=(t,),
            device_id_type=pl.DeviceIdType.MESH,
        )
        rdma.start()
        sends.append(rdma)

    lhs = x_ref[my]
    w_dma(0).wait()
    wt_cur = w_buf[0].astype(jnp.bfloat16)
    for t in range(N_TILES):
        p, c = divmod(t, N_CHUNKS)
        if c == 0 and p > 0:
            d = d_for_phase[p]
            recv = pltpu.make_async_remote_copy(
                src_ref=x_ref.at[k_order[p]],
                dst_ref=gath_ref.at[d - 1],
                send_sem=send_sems.at[d - 1],
                recv_sem=recv_sems.at[d - 1],
                device_id=(k_order[p],),
                device_id_type=pl.DeviceIdType.MESH,
            )
            recv.wait_recv()
            lhs = gath_ref[d - 1]
        wt_next = None
        if t + 1 < N_TILES:
            w_dma(t + 1).wait()
            wt_next = w_buf[(t + 1) % N_SLOTS].astype(jnp.bfloat16)
        nsl = pl.ds(c * N_CHUNK, N_CHUNK)
        part = jnp.dot(lhs, wt_cur, preferred_element_type=jnp.float32)
        if p == 0:
            out_ref[:, nsl] = part
        else:
            out_ref[:, nsl] = out_ref[:, nsl] + part
        if t + N_SLOTS - 1 < N_TILES:
            w_dma(t + N_SLOTS - 1).start()
        wt_cur = wt_next

    for rdma in sends:
        rdma.wait_send()


def kernel(x, w_mat):
    x16 = x.astype(jnp.bfloat16).reshape(N_DEV, M_BLK, K_BLK)
    return pl.pallas_call(
        _body,
        out_shape=jax.ShapeDtypeStruct((M_BLK, N_TOT), jnp.float32),
        in_specs=[
            pl.BlockSpec(memory_space=pltpu.VMEM),
            pl.BlockSpec(memory_space=pltpu.HBM),
        ],
        out_specs=pl.BlockSpec(memory_space=pltpu.VMEM),
        scratch_shapes=[
            pltpu.VMEM((N_DEV - 1, M_BLK, K_BLK), jnp.bfloat16),
            pltpu.VMEM((N_SLOTS, K_BLK, N_CHUNK), jnp.float32),
            pltpu.SemaphoreType.DMA((N_DEV - 1,)),
            pltpu.SemaphoreType.DMA((N_DEV - 1,)),
            pltpu.SemaphoreType.DMA((N_SLOTS,)),
        ],
        compiler_params=pltpu.CompilerParams(
            collective_id=0,
            vmem_limit_bytes=60 * 1024 * 1024,
        ),
    )(x16, w_mat)


# device time: 139776 ns/iter; 1.0704x vs baseline; 1.0704x over previous
import jax
import jax.numpy as jnp
from jax import lax
from jax.experimental import pallas as pl
from jax.experimental.pallas import tpu as pltpu

N_DEV = 4
M_BLK = 1024
K_BLK = 1024
N_TOT = 8192
N_CHUNK = 512
N_CHUNKS = N_TOT // N_CHUNK
N_TILES = N_DEV * N_CHUNKS
N_SLOTS = 4


def _body(x_ref, w_ref, out_ref, gath_ref, w_buf, send_sems, recv_sems, w_sems):
    my = lax.axis_index("i")

    d_for_phase = [None, 1, 3, 2]
    k_order = [my] + [lax.rem(my + (N_DEV - d), N_DEV) for d in d_for_phase[1:]]

    def w_dma(t):
        p, c = divmod(t, N_CHUNKS)
        return pltpu.make_async_copy(
            w_ref.at[pl.ds(k_order[p] * K_BLK, K_BLK),
                     pl.ds(c * N_CHUNK, N_CHUNK)],
            w_buf.at[t % N_SLOTS],
            w_sems.at[t % N_SLOTS],
        )

    for t in range(N_SLOTS - 1):
        w_dma(t).start()

    barrier = pltpu.get_barrier_semaphore()
    for d in range(1, N_DEV):
        peer = lax.rem(my + d, N_DEV)
        pl.semaphore_signal(
            barrier, inc=1,
            device_id=(peer,), device_id_type=pl.DeviceIdType.MESH,
        )
    pl.semaphore_wait(barrier, N_DEV - 1)

    sends = []
    for d in range(1, N_DEV):
        t = lax.rem(my + d, N_DEV)
        rdma = pltpu.make_async_remote_copy(
            src_ref=x_ref.at[t],
            dst_ref=gath_ref.at[d - 1],
            send_sem=send_sems.at[d - 1],
            recv_sem=recv_sems.at[d - 1],
            device_id=(t,),
            device_id_type=pl.DeviceIdType.MESH,
        )
        rdma.start()
        sends.append(rdma)

    lhs = x_ref[my].astype(jnp.float32)
    for t in range(N_TILES):
        p, c = divmod(t, N_CHUNKS)
        if c == 0 and p > 0:
            d = d_for_phase[p]
            recv = pltpu.make_async_remote_copy(
                src_ref=x_ref.at[k_order[p]],
                dst_ref=gath_ref.at[d - 1],
                send_sem=send_sems.at[d - 1],
                recv_sem=recv_sems.at[d - 1],
                device_id=(k_order[p],),
                device_id_type=pl.DeviceIdType.MESH,
            )
            recv.wait_recv()
            lhs = gath_ref[d - 1].astype(jnp.float32)
        w_dma(t).wait()
        nsl = pl.ds(c * N_CHUNK, N_CHUNK)
        part = lax.dot_general(
            lhs, w_buf[t % N_SLOTS],
            dimension_numbers=(((1,), (0,)), ((), ())),
            precision=lax.Precision.DEFAULT,
            preferred_element_type=jnp.float32,
        )
        if p == 0:
            out_ref[:, nsl] = part
        else:
            out_ref[:, nsl] = out_ref[:, nsl] + part
        if t + N_SLOTS - 1 < N_TILES:
            w_dma(t + N_SLOTS - 1).start()

    for rdma in sends:
        rdma.wait_send()


def kernel(x, w_mat):
    x16 = x.astype(jnp.bfloat16).reshape(N_DEV, M_BLK, K_BLK)
    return pl.pallas_call(
        _body,
        out_shape=jax.ShapeDtypeStruct((M_BLK, N_TOT), jnp.float32),
        in_specs=[
            pl.BlockSpec(memory_space=pltpu.VMEM),
            pl.BlockSpec(memory_space=pltpu.HBM),
        ],
        out_specs=pl.BlockSpec(memory_space=pltpu.VMEM),
        scratch_shapes=[
            pltpu.VMEM((N_DEV - 1, M_BLK, K_BLK), jnp.bfloat16),
            pltpu.VMEM((N_SLOTS, K_BLK, N_CHUNK), jnp.float32),
            pltpu.SemaphoreType.DMA((N_DEV - 1,)),
            pltpu.SemaphoreType.DMA((N_DEV - 1,)),
            pltpu.SemaphoreType.DMA((N_SLOTS,)),
        ],
        compiler_params=pltpu.CompilerParams(
            collective_id=0,
            vmem_limit_bytes=60 * 1024 * 1024,
        ),
    )(x16, w_mat)


# device time: 131144 ns/iter; 1.1408x vs baseline; 1.0658x over previous
import jax
import jax.numpy as jnp
from jax import lax
from jax.experimental import pallas as pl
from jax.experimental.pallas import tpu as pltpu

N_DEV = 4
M_BLK = 1024
K_BLK = 1024
N_TOT = 8192
N_CHUNK = 512
N_CHUNKS = N_TOT // N_CHUNK
N_TILES = N_DEV * N_CHUNKS
N_SLOTS = 4


def _body(x_ref, w_ref, out_ref, acc_ref, gath_ref, w_buf,
          send_sems, recv_sems, w_sems, out_sems):
    my = lax.axis_index("i")

    d_for_phase = [None, 1, 3, 2]
    k_order = [my] + [lax.rem(my + (N_DEV - d), N_DEV) for d in d_for_phase[1:]]

    def w_dma(t):
        p, c = divmod(t, N_CHUNKS)
        return pltpu.make_async_copy(
            w_ref.at[pl.ds(k_order[p] * K_BLK, K_BLK),
                     pl.ds(c * N_CHUNK, N_CHUNK)],
            w_buf.at[t % N_SLOTS],
            w_sems.at[t % N_SLOTS],
        )

    for t in range(N_SLOTS - 1):
        w_dma(t).start()

    barrier = pltpu.get_barrier_semaphore()
    for d in range(1, N_DEV):
        peer = lax.rem(my + d, N_DEV)
        pl.semaphore_signal(
            barrier, inc=1,
            device_id=(peer,), device_id_type=pl.DeviceIdType.MESH,
        )
    pl.semaphore_wait(barrier, N_DEV - 1)

    sends = []
    for d in range(1, N_DEV):
        t = lax.rem(my + d, N_DEV)
        rdma = pltpu.make_async_remote_copy(
            src_ref=x_ref.at[t],
            dst_ref=gath_ref.at[d - 1],
            send_sem=send_sems.at[d - 1],
            recv_sem=recv_sems.at[d - 1],
            device_id=(t,),
            device_id_type=pl.DeviceIdType.MESH,
        )
        rdma.start()
        sends.append(rdma)

    def out_dma(c):
        nsl = pl.ds(c * N_CHUNK, N_CHUNK)
        return pltpu.make_async_copy(
            acc_ref.at[:, nsl], out_ref.at[:, nsl],
            out_sems.at[c % N_SLOTS])

    lhs = x_ref[my].astype(jnp.float32)
    for t in range(N_TILES):
        p, c = divmod(t, N_CHUNKS)
        if c == 0 and p > 0:
            d = d_for_phase[p]
            recv = pltpu.make_async_remote_copy(
                src_ref=x_ref.at[k_order[p]],
                dst_ref=gath_ref.at[d - 1],
                send_sem=send_sems.at[d - 1],
                recv_sem=recv_sems.at[d - 1],
                device_id=(k_order[p],),
                device_id_type=pl.DeviceIdType.MESH,
            )
            recv.wait_recv()
            lhs = gath_ref[d - 1].astype(jnp.float32)
        w_dma(t).wait()
        nsl = pl.ds(c * N_CHUNK, N_CHUNK)
        part = lax.dot_general(
            lhs, w_buf[t % N_SLOTS],
            dimension_numbers=(((1,), (0,)), ((), ())),
            precision=lax.Precision.DEFAULT,
            preferred_element_type=jnp.float32,
        )
        if p == 0:
            acc_ref[:, nsl] = part
        elif p < N_DEV - 1:
            acc_ref[:, nsl] = acc_ref[:, nsl] + part
        else:
            acc_ref[:, nsl] = acc_ref[:, nsl] + part
            if c >= N_SLOTS:
                out_dma(c - N_SLOTS).wait()
            out_dma(c).start()
        if t + N_SLOTS - 1 < N_TILES:
            w_dma(t + N_SLOTS - 1).start()

    for c in range(N_CHUNKS - N_SLOTS, N_CHUNKS):
        out_dma(c).wait()
    for rdma in sends:
        rdma.wait_send()


def kernel(x, w_mat):
    x16 = x.astype(jnp.bfloat16).reshape(N_DEV, M_BLK, K_BLK)
    return pl.pallas_call(
        _body,
        out_shape=jax.ShapeDtypeStruct((M_BLK, N_TOT), jnp.float32),
        in_specs=[
            pl.BlockSpec(memory_space=pltpu.VMEM),
            pl.BlockSpec(memory_space=pltpu.HBM),
        ],
        out_specs=pl.BlockSpec(memory_space=pltpu.HBM),
        scratch_shapes=[
            pltpu.VMEM((M_BLK, N_TOT), jnp.float32),
            pltpu.VMEM((N_DEV - 1, M_BLK, K_BLK), jnp.bfloat16),
            pltpu.VMEM((N_SLOTS, K_BLK, N_CHUNK), jnp.float32),
            pltpu.SemaphoreType.DMA((N_DEV - 1,)),
            pltpu.SemaphoreType.DMA((N_DEV - 1,)),
            pltpu.SemaphoreType.DMA((N_SLOTS,)),
            pltpu.SemaphoreType.DMA((N_SLOTS,)),
        ],
        compiler_params=pltpu.CompilerParams(
            collective_id=0,
            vmem_limit_bytes=60 * 1024 * 1024,
        ),
    )(x16, w_mat)
